# baseline (speedup 1.0000x reference)
import functools
import math

import jax
import jax.numpy as jnp
from jax import lax
from jax.experimental import pallas as pl
from jax.experimental.pallas import tpu as pltpu
from jax.experimental.pallas import tpu_sc as plsc

_B, _T, _D, _H, _E, _K = 1, 2048, 768, 12, 8, 2
_DFF = 4 * _D
_N = _B * _T
_cc = math.floor(_K * 1.25 * _N / _E)
_cc += _cc % 2
_CAP = max(_cc, 4)
_HD = _D // _H
_SLOTS = _E * _CAP
_NPAIR = _K * _N

_BQ = 512
_BR = 256
_F32 = jnp.float32


def _f32dot(a, b):
    return jnp.dot(a, b, preferred_element_type=_F32)


def _f32dot_nt(a, b):
    return lax.dot_general(a, b, (((1,), (1,)), ((), ())),
                           preferred_element_type=_F32)


def _gelu(x):
    return 0.5 * x * (1.0 + lax.erf(x * (1.0 / math.sqrt(2.0))))


def _ln_qkv_body(x_ref, g_ref, b_ref, w_ref, bias_ref, q_ref, k_ref, v_ref):
    x = x_ref[...]
    m = jnp.mean(x, axis=-1, keepdims=True)
    c = x - m
    v = jnp.mean(c * c, axis=-1, keepdims=True)
    h = c / jnp.sqrt(v + 1e-5) * g_ref[...] + b_ref[...]
    qkv = _f32dot_nt(h, w_ref[...]) + bias_ref[...]
    for hh in range(_H):
        q_ref[hh] = qkv[:, hh * _HD:(hh + 1) * _HD]
        k_ref[hh] = qkv[:, _D + hh * _HD:_D + (hh + 1) * _HD]
        v_ref[hh] = qkv[:, 2 * _D + hh * _HD:2 * _D + (hh + 1) * _HD]


def _ln_qkv(x2d, g, b, w, bias):
    hs = jax.ShapeDtypeStruct((_H, _T, _HD), _F32)
    return pl.pallas_call(
        _ln_qkv_body,
        grid=(_N // _BR,),
        in_specs=[
            pl.BlockSpec((_BR, _D), lambda i: (i, 0)),
            pl.BlockSpec((1, _D), lambda i: (0, 0)),
            pl.BlockSpec((1, _D), lambda i: (0, 0)),
            pl.BlockSpec((3 * _D, _D), lambda i: (0, 0)),
            pl.BlockSpec((1, 3 * _D), lambda i: (0, 0)),
        ],
        out_specs=[pl.BlockSpec((_H, _BR, _HD), lambda i: (0, i, 0))] * 3,
        out_shape=[hs, hs, hs],
    )(x2d, g.reshape(1, _D), b.reshape(1, _D), w, bias.reshape(1, 3 * _D))


def _attn_body(q_ref, k_ref, v_ref, o_ref):
    qi = pl.program_id(1)
    q = q_ref[0]
    scale = 1.0 / math.sqrt(_HD)

    def tile(ki, carry, masked):
        m, l, acc = carry
        k = k_ref[0, pl.ds(ki * _BQ, _BQ), :]
        s = _f32dot_nt(q, k) * scale
        if masked:
            rows = lax.broadcasted_iota(jnp.int32, (_BQ, _BQ), 0)
            cols = lax.broadcasted_iota(jnp.int32, (_BQ, _BQ), 1)
            s = jnp.where(rows >= cols, s, -jnp.inf)
        mn = jnp.maximum(m, jnp.max(s, axis=1, keepdims=True))
        p = jnp.exp(s - mn)
        alpha = jnp.exp(m - mn)
        l2 = l * alpha + jnp.sum(p, axis=1, keepdims=True)
        v = v_ref[0, pl.ds(ki * _BQ, _BQ), :]
        acc2 = acc * alpha + _f32dot(p, v)
        return mn, l2, acc2

    m0 = jnp.full((_BQ, 1), -jnp.inf, _F32)
    l0 = jnp.zeros((_BQ, 1), _F32)
    a0 = jnp.zeros((_BQ, _HD), _F32)
    carry = lax.fori_loop(0, qi, lambda ki, c: tile(ki, c, False),
                          (m0, l0, a0))
    m, l, acc = tile(qi, carry, True)
    o_ref[0] = acc / l


def _attention(q, k, v):
    return pl.pallas_call(
        _attn_body,
        grid=(_H, _T // _BQ),
        in_specs=[
            pl.BlockSpec((1, _BQ, _HD), lambda h, i: (h, i, 0)),
            pl.BlockSpec((1, _T, _HD), lambda h, i: (h, 0, 0)),
            pl.BlockSpec((1, _T, _HD), lambda h, i: (h, 0, 0)),
        ],
        out_specs=pl.BlockSpec((1, _BQ, _HD), lambda h, i: (h, i, 0)),
        out_shape=jax.ShapeDtypeStruct((_H, _T, _HD), _F32),
    )(q, k, v)


def _proj_body(y_ref, x_ref, w_ref, b_ref, g2_ref, b2_ref, wg_ref,
               x1_ref, h2_ref, lg_ref):
    y = jnp.concatenate([y_ref[hh] for hh in range(_H)], axis=1)
    x1 = x_ref[...] + _f32dot_nt(y, w_ref[...]) + b_ref[...]
    x1_ref[...] = x1
    m = jnp.mean(x1, axis=-1, keepdims=True)
    c = x1 - m
    v = jnp.mean(c * c, axis=-1, keepdims=True)
    h2 = c / jnp.sqrt(v + 1e-5) * g2_ref[...] + b2_ref[...]
    h2_ref[...] = h2
    lg_ref[...] = _f32dot_nt(h2, wg_ref[...])


def _proj_ln2_logits(y, x2d, w, b, g2, b2, wg):
    return pl.pallas_call(
        _proj_body,
        grid=(_N // _BR,),
        in_specs=[
            pl.BlockSpec((_H, _BR, _HD), lambda i: (0, i, 0)),
            pl.BlockSpec((_BR, _D), lambda i: (i, 0)),
            pl.BlockSpec((_D, _D), lambda i: (0, 0)),
            pl.BlockSpec((1, _D), lambda i: (0, 0)),
            pl.BlockSpec((1, _D), lambda i: (0, 0)),
            pl.BlockSpec((1, _D), lambda i: (0, 0)),
            pl.BlockSpec((_E, _D), lambda i: (0, 0)),
        ],
        out_specs=[
            pl.BlockSpec((_BR, _D), lambda i: (i, 0)),
            pl.BlockSpec((_BR, _D), lambda i: (i, 0)),
            pl.BlockSpec((_BR, _E), lambda i: (i, 0)),
        ],
        out_shape=[
            jax.ShapeDtypeStruct((_N, _D), _F32),
            jax.ShapeDtypeStruct((_N, _D), _F32),
            jax.ShapeDtypeStruct((_N, _E), _F32),
        ],
    )(y, x2d, w, b.reshape(1, _D), g2.reshape(1, _D), b2.reshape(1, _D), wg)


_RCH = 256


def _router_body(nworkers, lg_ref, rt_ref, cnt_ref, oh0_ref, oh1_ref, c0_ref, c1_ref):
    l = lg_ref[...]
    eidx = lax.broadcasted_iota(jnp.int32, (_N, _E), 1)
    m0 = jnp.max(l, axis=1, keepdims=True)
    e0 = jnp.min(jnp.where(l == m0, eidx, _E), axis=1, keepdims=True)
    oh0 = (eidx == e0)
    lm = jnp.where(oh0, -jnp.inf, l)
    m1 = jnp.max(lm, axis=1, keepdims=True)
    e1 = jnp.min(jnp.where(lm == m1, eidx, _E), axis=1, keepdims=True)
    oh1 = (eidx == e1)
    t = jnp.exp(m1 - m0)
    w0 = 1.0 / (1.0 + t)
    w1 = t * w0
    oh0_ref[...] = oh0.astype(_F32)
    oh1_ref[...] = oh1.astype(_F32)

    tril = (lax.broadcasted_iota(jnp.int32, (_RCH, _RCH), 0)
            >= lax.broadcasted_iota(jnp.int32, (_RCH, _RCH), 1)).astype(_F32)

    def scan(oh_ref, cum_ref):
        def chunk(i, tot):
            oh = oh_ref[pl.ds(i * _RCH, _RCH), :]
            cum_ref[pl.ds(i * _RCH, _RCH), :] = _f32dot(tril, oh) + tot
            return tot + jnp.sum(oh, axis=0, keepdims=True)
        return lax.fori_loop(0, _N // _RCH, chunk, jnp.zeros((1, _E), _F32))

    tot0 = scan(oh0_ref, c0_ref)
    tot1 = scan(oh1_ref, c1_ref)
    cnt_ref[...] = jnp.minimum(tot0 + tot1, float(_CAP)).astype(jnp.int32)

    oh0f = oh0_ref[...]
    oh1f = oh1_ref[...]
    p0 = jnp.sum(oh0f * c0_ref[...], axis=1, keepdims=True) - 1.0
    p1 = jnp.sum(oh1f * (c1_ref[...] + tot0), axis=1, keepdims=True) - 1.0

    ppw = _NPAIR // nworkers
    nidx = lax.broadcasted_iota(jnp.int32, (_N, 1), 0)
    dump0 = (_SLOTS + nidx // ppw).astype(_F32)
    dump1 = (_SLOTS + (_N + nidx) // ppw).astype(_F32)
    keep0 = p0 < _CAP
    keep1 = p1 < _CAP
    e0f = e0.astype(_F32)
    e1f = e1.astype(_F32)
    d0 = jnp.where(keep0, e0f * _CAP + p0, dump0)
    d1 = jnp.where(keep1, e1f * _CAP + p1, dump1)
    w0 = jnp.where(keep0, w0, 0.0)
    w1 = jnp.where(keep1, w1, 0.0)
    z = jnp.zeros((_N, 4), _F32)
    rt_ref[...] = jnp.concatenate([d0, d1, w0, w1, z], axis=1)


def _router(logits, nworkers):
    return pl.pallas_call(
        functools.partial(_router_body, nworkers),
        grid=(1,),
        in_specs=[pl.BlockSpec((_N, _E), lambda i: (0, 0))],
        out_specs=[
            pl.BlockSpec((_N, 8), lambda i: (0, 0)),
            pl.BlockSpec((1, _E), lambda i: (0, 0)),
        ],
        out_shape=[
            jax.ShapeDtypeStruct((_N, 8), _F32),
            jax.ShapeDtypeStruct((1, _E), jnp.int32),
        ],
        scratch_shapes=[pltpu.VMEM((_N, _E), _F32) for _ in range(4)],
    )(logits)


def _sc_meshinfo():
    info = plsc.get_sparse_core_info()
    return info.num_cores, info.num_subcores


def _dispatch_sc(h2, slots):
    nc, ns = _sc_meshinfo()
    nw = nc * ns
    ppw = _NPAIR // nw
    mesh = plsc.VectorSubcoreMesh(core_axis_name="c", subcore_axis_name="s")

    @functools.partial(
        pl.kernel, mesh=mesh,
        out_type=jax.ShapeDtypeStruct((_SLOTS + nw, _D), _F32),
        scratch_types=[
            pltpu.VMEM((ppw,), jnp.int32),
            pltpu.VMEM((ppw, _D), _F32),
        ],
    )
    def k(h2_hbm, slots_hbm, out_hbm, idx_v, rows_v):
        wid = lax.axis_index("s") * nc + lax.axis_index("c")
        tok = (wid * ppw) % _N
        pltpu.sync_copy(slots_hbm.at[wid], idx_v)
        pltpu.sync_copy(h2_hbm.at[pl.ds(tok, ppw)], rows_v)
        pltpu.sync_copy(rows_v, out_hbm.at[idx_v])

    return k(h2, slots)


def _gather_sc(o_flat, slots):
    nc, ns = _sc_meshinfo()
    nw = nc * ns
    ppw = _NPAIR // nw
    mesh = plsc.VectorSubcoreMesh(core_axis_name="c", subcore_axis_name="s")

    @functools.partial(
        pl.kernel, mesh=mesh,
        out_type=jax.ShapeDtypeStruct((_NPAIR, _D), _F32),
        scratch_types=[
            pltpu.VMEM((ppw,), jnp.int32),
            pltpu.VMEM((ppw, _D), _F32),
            pltpu.SemaphoreType.DMA,
        ],
    )
    def k(o_hbm, slots_hbm, out_hbm, idx_v, rows_v, sem):
        wid = lax.axis_index("s") * nc + lax.axis_index("c")
        pltpu.sync_copy(slots_hbm.at[wid], idx_v)
        pltpu.async_copy(o_hbm.at[idx_v], rows_v, sem).wait()
        pltpu.sync_copy(rows_v, out_hbm.at[pl.ds(wid * ppw, ppw)])

    return k(o_flat, slots)


_MROW = 128


def _mlp_body(cnt_ref, disp_ref, wfc_ref, bfc_ref, wpr_ref, bpr_ref,
              o_ref, acc_ref):
    e = pl.program_id(0)
    j = pl.program_id(1)
    nj = _DFF // _D
    ce = cnt_ref[e]
    for c in range(_CAP // _MROW):
        @pl.when(c * _MROW < ce)
        def _(c=c):
            rows = pl.ds(c * _MROW, _MROW)
            a = _gelu(_f32dot(disp_ref[rows, :], wfc_ref[0]) + bfc_ref[0])
            part = _f32dot(a, wpr_ref[0])

            @pl.when(j == 0)
            def _():
                acc_ref[rows, :] = part

            @pl.when(j > 0)
            def _():
                acc_ref[rows, :] += part

            @pl.when(j == nj - 1)
            def _():
                o_ref[rows, :] = acc_ref[rows, :] + bpr_ref[0]


def _expert_mlps(disp, c_fc, fc_bias, c_proj_e, proj_bias, cnt, nrows):
    nj = _DFF // _D
    return pl.pallas_call(
        _mlp_body,
        grid=(_E, nj),
        in_specs=[
            pl.BlockSpec(memory_space=pltpu.SMEM),
            pl.BlockSpec((_CAP, _D), lambda e, j: (e, 0)),
            pl.BlockSpec((1, _D, _D), lambda e, j: (e, 0, j)),
            pl.BlockSpec((1, 1, _D), lambda e, j: (e, 0, j)),
            pl.BlockSpec((1, _D, _D), lambda e, j: (e, j, 0)),
            pl.BlockSpec((1, 1, _D), lambda e, j: (e, 0, 0)),
        ],
        out_specs=pl.BlockSpec((_CAP, _D), lambda e, j: (e, 0)),
        out_shape=jax.ShapeDtypeStruct((nrows, _D), _F32),
        scratch_shapes=[pltpu.VMEM((_CAP, _D), _F32)],
    )(cnt, disp, c_fc, fc_bias, c_proj_e, proj_bias)


def _combine_body(x1_ref, g0_ref, g1_ref, w0_ref, w1_ref, o_ref):
    w0 = w0_ref[...]
    w1 = w1_ref[...]
    o_ref[...] = (x1_ref[...]
                  + jnp.where(w0 == 0.0, 0.0, w0 * g0_ref[...])
                  + jnp.where(w1 == 0.0, 0.0, w1 * g1_ref[...]))


def _combine(x1, g, w0, w1):
    nb = _N // _BR
    return pl.pallas_call(
        _combine_body,
        grid=(nb,),
        in_specs=[
            pl.BlockSpec((_BR, _D), lambda i: (i, 0)),
            pl.BlockSpec((_BR, _D), lambda i: (i, 0)),
            pl.BlockSpec((_BR, _D), lambda i, _nb=nb: (i + _nb, 0)),
            pl.BlockSpec((_BR, 1), lambda i: (i, 0)),
            pl.BlockSpec((_BR, 1), lambda i: (i, 0)),
        ],
        out_specs=pl.BlockSpec((_BR, _D), lambda i: (i, 0)),
        out_shape=jax.ShapeDtypeStruct((_N, _D), _F32),
    )(x1, g, g, w0, w1)


def kernel(x, ln1_g, ln1_b, c_attn_w, c_attn_b, c_proj_w, c_proj_b,
           ln2_g, ln2_b, w_g, c_fc, fc_bias, c_proj_e, proj_bias):
    nc, ns = _sc_meshinfo()
    nw = nc * ns

    x2d = x.reshape(_N, _D)
    q, k, v = _ln_qkv(x2d, ln1_g, ln1_b, c_attn_w, c_attn_b)
    y = _attention(q, k, v)

    x1, h2, logits = _proj_ln2_logits(y, x2d, c_proj_w, c_proj_b,
                                      ln2_g, ln2_b, w_g)

    rt, cnt = _router(logits, nw)
    slots = jnp.concatenate([rt[:, 0], rt[:, 1]]).astype(jnp.int32)
    slots = slots.reshape(nw, _NPAIR // nw)
    w0 = rt[:, 2:3]
    w1 = rt[:, 3:4]

    disp = _dispatch_sc(h2, slots)
    o_flat = _expert_mlps(disp, c_fc, fc_bias, c_proj_e, proj_bias,
                          cnt.reshape(_E), _SLOTS + nw)
    g = _gather_sc(o_flat, slots)
    out = _combine(x1, g, w0, w1)
    return out.reshape(_B, _T, _D)


# speedup vs baseline: 1.1140x; 1.1140x over previous
import functools
import math

import jax
import jax.numpy as jnp
from jax import lax
from jax.experimental import pallas as pl
from jax.experimental.pallas import tpu as pltpu
from jax.experimental.pallas import tpu_sc as plsc

_B, _T, _D, _H, _E, _K = 1, 2048, 768, 12, 8, 2
_DFF = 4 * _D
_N = _B * _T
_cc = math.floor(_K * 1.25 * _N / _E)
_cc += _cc % 2
_CAP = max(_cc, 4)
_HD = _D // _H
_SLOTS = _E * _CAP
_NPAIR = _K * _N

_BQ = 512
_BR = 256
_F32 = jnp.float32


def _f32dot(a, b):
    return jnp.dot(a, b, preferred_element_type=_F32)


def _f32dot_nt(a, b):
    return lax.dot_general(a, b, (((1,), (1,)), ((), ())),
                           preferred_element_type=_F32)


def _gelu(x):
    return 0.5 * x * (1.0 + lax.erf(x * (1.0 / math.sqrt(2.0))))


def _ln_qkv_body(x_ref, g_ref, b_ref, w_ref, bias_ref, q_ref, k_ref, v_ref):
    x = x_ref[...]
    m = jnp.mean(x, axis=-1, keepdims=True)
    c = x - m
    v = jnp.mean(c * c, axis=-1, keepdims=True)
    h = c / jnp.sqrt(v + 1e-5) * g_ref[...] + b_ref[...]
    qkv = _f32dot_nt(h, w_ref[...]) + bias_ref[...]
    for hh in range(_H):
        q_ref[hh] = qkv[:, hh * _HD:(hh + 1) * _HD]
        k_ref[hh] = qkv[:, _D + hh * _HD:_D + (hh + 1) * _HD]
        v_ref[hh] = qkv[:, 2 * _D + hh * _HD:2 * _D + (hh + 1) * _HD]


def _ln_qkv(x2d, g, b, w, bias):
    hs = jax.ShapeDtypeStruct((_H, _T, _HD), _F32)
    return pl.pallas_call(
        _ln_qkv_body,
        grid=(_N // _BR,),
        in_specs=[
            pl.BlockSpec((_BR, _D), lambda i: (i, 0)),
            pl.BlockSpec((1, _D), lambda i: (0, 0)),
            pl.BlockSpec((1, _D), lambda i: (0, 0)),
            pl.BlockSpec((3 * _D, _D), lambda i: (0, 0)),
            pl.BlockSpec((1, 3 * _D), lambda i: (0, 0)),
        ],
        out_specs=[pl.BlockSpec((_H, _BR, _HD), lambda i: (0, i, 0))] * 3,
        out_shape=[hs, hs, hs],
    )(x2d, g.reshape(1, _D), b.reshape(1, _D), w, bias.reshape(1, 3 * _D))


def _attn_body(q_ref, k_ref, v_ref, o_ref):
    qi = pl.program_id(1)
    q = q_ref[0]
    scale = 1.0 / math.sqrt(_HD)

    def tile(ki, carry, masked):
        m, l, acc = carry
        k = k_ref[0, pl.ds(ki * _BQ, _BQ), :]
        s = _f32dot_nt(q, k) * scale
        if masked:
            rows = lax.broadcasted_iota(jnp.int32, (_BQ, _BQ), 0)
            cols = lax.broadcasted_iota(jnp.int32, (_BQ, _BQ), 1)
            s = jnp.where(rows >= cols, s, -jnp.inf)
        mn = jnp.maximum(m, jnp.max(s, axis=1, keepdims=True))
        p = jnp.exp(s - mn)
        alpha = jnp.exp(m - mn)
        l2 = l * alpha + jnp.sum(p, axis=1, keepdims=True)
        v = v_ref[0, pl.ds(ki * _BQ, _BQ), :]
        acc2 = acc * alpha + _f32dot(p, v)
        return mn, l2, acc2

    m0 = jnp.full((_BQ, 1), -jnp.inf, _F32)
    l0 = jnp.zeros((_BQ, 1), _F32)
    a0 = jnp.zeros((_BQ, _HD), _F32)
    carry = lax.fori_loop(0, qi, lambda ki, c: tile(ki, c, False),
                          (m0, l0, a0))
    m, l, acc = tile(qi, carry, True)
    o_ref[0] = acc / l


def _attention(q, k, v):
    return pl.pallas_call(
        _attn_body,
        grid=(_H, _T // _BQ),
        in_specs=[
            pl.BlockSpec((1, _BQ, _HD), lambda h, i: (h, i, 0)),
            pl.BlockSpec((1, _T, _HD), lambda h, i: (h, 0, 0)),
            pl.BlockSpec((1, _T, _HD), lambda h, i: (h, 0, 0)),
        ],
        out_specs=pl.BlockSpec((1, _BQ, _HD), lambda h, i: (h, i, 0)),
        out_shape=jax.ShapeDtypeStruct((_H, _T, _HD), _F32),
    )(q, k, v)


def _proj_body(y_ref, x_ref, w_ref, b_ref, g2_ref, b2_ref, wg_ref,
               x1_ref, h2_ref, lg_ref):
    y = jnp.concatenate([y_ref[hh] for hh in range(_H)], axis=1)
    x1 = x_ref[...] + _f32dot_nt(y, w_ref[...]) + b_ref[...]
    x1_ref[...] = x1
    m = jnp.mean(x1, axis=-1, keepdims=True)
    c = x1 - m
    v = jnp.mean(c * c, axis=-1, keepdims=True)
    h2 = c / jnp.sqrt(v + 1e-5) * g2_ref[...] + b2_ref[...]
    h2_ref[...] = h2
    lg_ref[...] = _f32dot_nt(h2, wg_ref[...])


def _proj_ln2_logits(y, x2d, w, b, g2, b2, wg):
    return pl.pallas_call(
        _proj_body,
        grid=(_N // _BR,),
        in_specs=[
            pl.BlockSpec((_H, _BR, _HD), lambda i: (0, i, 0)),
            pl.BlockSpec((_BR, _D), lambda i: (i, 0)),
            pl.BlockSpec((_D, _D), lambda i: (0, 0)),
            pl.BlockSpec((1, _D), lambda i: (0, 0)),
            pl.BlockSpec((1, _D), lambda i: (0, 0)),
            pl.BlockSpec((1, _D), lambda i: (0, 0)),
            pl.BlockSpec((_E, _D), lambda i: (0, 0)),
        ],
        out_specs=[
            pl.BlockSpec((_BR, _D), lambda i: (i, 0)),
            pl.BlockSpec((_BR, _D), lambda i: (i, 0)),
            pl.BlockSpec((_BR, _E), lambda i: (i, 0)),
        ],
        out_shape=[
            jax.ShapeDtypeStruct((_N, _D), _F32),
            jax.ShapeDtypeStruct((_N, _D), _F32),
            jax.ShapeDtypeStruct((_N, _E), _F32),
        ],
    )(y, x2d, w, b.reshape(1, _D), g2.reshape(1, _D), b2.reshape(1, _D), wg)


_RCH = 256


def _router_body(nworkers, lg_ref, rt_ref, cnt_ref, oh0_ref, oh1_ref, c0_ref, c1_ref):
    l = lg_ref[...]
    eidx = lax.broadcasted_iota(jnp.int32, (_N, _E), 1)
    m0 = jnp.max(l, axis=1, keepdims=True)
    e0 = jnp.min(jnp.where(l == m0, eidx, _E), axis=1, keepdims=True)
    oh0 = (eidx == e0)
    lm = jnp.where(oh0, -jnp.inf, l)
    m1 = jnp.max(lm, axis=1, keepdims=True)
    e1 = jnp.min(jnp.where(lm == m1, eidx, _E), axis=1, keepdims=True)
    oh1 = (eidx == e1)
    t = jnp.exp(m1 - m0)
    w0 = 1.0 / (1.0 + t)
    w1 = t * w0
    oh0_ref[...] = oh0.astype(_F32)
    oh1_ref[...] = oh1.astype(_F32)

    tril = (lax.broadcasted_iota(jnp.int32, (_RCH, _RCH), 0)
            >= lax.broadcasted_iota(jnp.int32, (_RCH, _RCH), 1)).astype(_F32)

    def scan(oh_ref, cum_ref):
        def chunk(i, tot):
            oh = oh_ref[pl.ds(i * _RCH, _RCH), :]
            cum_ref[pl.ds(i * _RCH, _RCH), :] = _f32dot(tril, oh) + tot
            return tot + jnp.sum(oh, axis=0, keepdims=True)
        return lax.fori_loop(0, _N // _RCH, chunk, jnp.zeros((1, _E), _F32))

    tot0 = scan(oh0_ref, c0_ref)
    tot1 = scan(oh1_ref, c1_ref)
    cnt_ref[...] = jnp.minimum(tot0 + tot1, float(_CAP)).astype(jnp.int32)

    oh0f = oh0_ref[...]
    oh1f = oh1_ref[...]
    p0 = jnp.sum(oh0f * c0_ref[...], axis=1, keepdims=True) - 1.0
    p1 = jnp.sum(oh1f * (c1_ref[...] + tot0), axis=1, keepdims=True) - 1.0

    ppw = _NPAIR // nworkers
    nidx = lax.broadcasted_iota(jnp.int32, (_N, 1), 0)
    dump0 = (_SLOTS + nidx // ppw).astype(_F32)
    dump1 = (_SLOTS + (_N + nidx) // ppw).astype(_F32)
    keep0 = p0 < _CAP
    keep1 = p1 < _CAP
    e0f = e0.astype(_F32)
    e1f = e1.astype(_F32)
    d0 = jnp.where(keep0, e0f * _CAP + p0, dump0)
    d1 = jnp.where(keep1, e1f * _CAP + p1, dump1)
    w0 = jnp.where(keep0, w0, 0.0)
    w1 = jnp.where(keep1, w1, 0.0)
    z = jnp.zeros((_N, 4), _F32)
    rt_ref[...] = jnp.concatenate([d0, d1, w0, w1, z], axis=1)


def _router(logits, nworkers):
    return pl.pallas_call(
        functools.partial(_router_body, nworkers),
        grid=(1,),
        in_specs=[pl.BlockSpec((_N, _E), lambda i: (0, 0))],
        out_specs=[
            pl.BlockSpec((_N, 8), lambda i: (0, 0)),
            pl.BlockSpec((1, _E), lambda i: (0, 0)),
        ],
        out_shape=[
            jax.ShapeDtypeStruct((_N, 8), _F32),
            jax.ShapeDtypeStruct((1, _E), jnp.int32),
        ],
        scratch_shapes=[pltpu.VMEM((_N, _E), _F32) for _ in range(4)],
    )(logits)


def _sc_meshinfo():
    info = plsc.get_sparse_core_info()
    return info.num_cores, info.num_subcores


def _dispatch_sc(h2, slots):
    nc, ns = _sc_meshinfo()
    nw = nc * ns
    ppw = _NPAIR // nw
    mesh = plsc.VectorSubcoreMesh(core_axis_name="c", subcore_axis_name="s")

    @functools.partial(
        pl.kernel, mesh=mesh,
        out_type=jax.ShapeDtypeStruct((_SLOTS + nw, _D), _F32),
        scratch_types=[
            pltpu.VMEM((ppw,), jnp.int32),
            pltpu.VMEM((ppw, _D), _F32),
        ],
    )
    def k(h2_hbm, slots_hbm, out_hbm, idx_v, rows_v):
        wid = lax.axis_index("s") * nc + lax.axis_index("c")
        tok = (wid * ppw) % _N
        pltpu.sync_copy(slots_hbm.at[wid], idx_v)
        pltpu.sync_copy(h2_hbm.at[pl.ds(tok, ppw)], rows_v)
        pltpu.sync_copy(rows_v, out_hbm.at[idx_v])

    return k(h2, slots)


def _gather_sc(o_flat, slots):
    nc, ns = _sc_meshinfo()
    nw = nc * ns
    ppw = _NPAIR // nw
    mesh = plsc.VectorSubcoreMesh(core_axis_name="c", subcore_axis_name="s")

    @functools.partial(
        pl.kernel, mesh=mesh,
        out_type=jax.ShapeDtypeStruct((_NPAIR, _D), _F32),
        scratch_types=[
            pltpu.VMEM((ppw,), jnp.int32),
            pltpu.VMEM((ppw, _D), _F32),
            pltpu.SemaphoreType.DMA,
        ],
    )
    def k(o_hbm, slots_hbm, out_hbm, idx_v, rows_v, sem):
        wid = lax.axis_index("s") * nc + lax.axis_index("c")
        pltpu.sync_copy(slots_hbm.at[wid], idx_v)
        pltpu.async_copy(o_hbm.at[idx_v], rows_v, sem).wait()
        pltpu.sync_copy(rows_v, out_hbm.at[pl.ds(wid * ppw, ppw)])

    return k(o_flat, slots)


def _mlp_body(disp_ref, wfc_ref, bfc_ref, wpr_ref, bpr_ref, o_ref, acc_ref):
    j = pl.program_id(1)
    a = _gelu(_f32dot(disp_ref[...], wfc_ref[0]) + bfc_ref[0])
    part = _f32dot(a, wpr_ref[0])

    @pl.when(j == 0)
    def _():
        acc_ref[...] = part

    @pl.when(j > 0)
    def _():
        acc_ref[...] += part

    @pl.when(j == _DFF // _D - 1)
    def _():
        o_ref[...] = acc_ref[...] + bpr_ref[0]


def _expert_mlps(disp, c_fc, fc_bias, c_proj_e, proj_bias, nrows):
    nj = _DFF // _D
    return pl.pallas_call(
        _mlp_body,
        grid=(_E, nj),
        in_specs=[
            pl.BlockSpec((_CAP, _D), lambda e, j: (e, 0)),
            pl.BlockSpec((1, _D, _D), lambda e, j: (e, 0, j)),
            pl.BlockSpec((1, 1, _D), lambda e, j: (e, 0, j)),
            pl.BlockSpec((1, _D, _D), lambda e, j: (e, j, 0)),
            pl.BlockSpec((1, 1, _D), lambda e, j: (e, 0, 0)),
        ],
        out_specs=pl.BlockSpec((_CAP, _D), lambda e, j: (e, 0)),
        out_shape=jax.ShapeDtypeStruct((nrows, _D), _F32),
        scratch_shapes=[pltpu.VMEM((_CAP, _D), _F32)],
    )(disp, c_fc, fc_bias, c_proj_e, proj_bias)


def _combine_body(x1_ref, g0_ref, g1_ref, w0_ref, w1_ref, o_ref):
    w0 = w0_ref[...]
    w1 = w1_ref[...]
    o_ref[...] = (x1_ref[...]
                  + jnp.where(w0 == 0.0, 0.0, w0 * g0_ref[...])
                  + jnp.where(w1 == 0.0, 0.0, w1 * g1_ref[...]))


def _combine(x1, g, w0, w1):
    nb = _N // _BR
    return pl.pallas_call(
        _combine_body,
        grid=(nb,),
        in_specs=[
            pl.BlockSpec((_BR, _D), lambda i: (i, 0)),
            pl.BlockSpec((_BR, _D), lambda i: (i, 0)),
            pl.BlockSpec((_BR, _D), lambda i, _nb=nb: (i + _nb, 0)),
            pl.BlockSpec((_BR, 1), lambda i: (i, 0)),
            pl.BlockSpec((_BR, 1), lambda i: (i, 0)),
        ],
        out_specs=pl.BlockSpec((_BR, _D), lambda i: (i, 0)),
        out_shape=jax.ShapeDtypeStruct((_N, _D), _F32),
    )(x1, g, g, w0, w1)


def kernel(x, ln1_g, ln1_b, c_attn_w, c_attn_b, c_proj_w, c_proj_b,
           ln2_g, ln2_b, w_g, c_fc, fc_bias, c_proj_e, proj_bias):
    nc, ns = _sc_meshinfo()
    nw = nc * ns

    x2d = x.reshape(_N, _D)
    q, k, v = _ln_qkv(x2d, ln1_g, ln1_b, c_attn_w, c_attn_b)
    y = _attention(q, k, v)

    x1, h2, logits = _proj_ln2_logits(y, x2d, c_proj_w, c_proj_b,
                                      ln2_g, ln2_b, w_g)

    rt, cnt = _router(logits, nw)
    slots = jnp.concatenate([rt[:, 0], rt[:, 1]]).astype(jnp.int32)
    slots = slots.reshape(nw, _NPAIR // nw)
    w0 = rt[:, 2:3]
    w1 = rt[:, 3:4]

    disp = _dispatch_sc(h2, slots)
    o_flat = _expert_mlps(disp, c_fc, fc_bias, c_proj_e, proj_bias,
                          _SLOTS + nw)
    g = _gather_sc(o_flat, slots)
    out = _combine(x1, g, w0, w1)
    return out.reshape(_B, _T, _D)


# speedup vs baseline: 1.2175x; 1.0930x over previous
import functools
import math

import jax
import jax.numpy as jnp
from jax import lax
from jax.experimental import pallas as pl
from jax.experimental.pallas import tpu as pltpu
from jax.experimental.pallas import tpu_sc as plsc

_B, _T, _D, _H, _E, _K = 1, 2048, 768, 12, 8, 2
_DFF = 4 * _D
_N = _B * _T
_cc = math.floor(_K * 1.25 * _N / _E)
_cc += _cc % 2
_CAP = max(_cc, 4)
_HD = _D // _H
_SLOTS = _E * _CAP
_NPAIR = _K * _N

_BQ = 1024
_BR = 256
_F32 = jnp.float32


def _f32dot(a, b):
    return jnp.dot(a, b, preferred_element_type=_F32)


def _f32dot_nt(a, b):
    return lax.dot_general(a, b, (((1,), (1,)), ((), ())),
                           preferred_element_type=_F32)


def _gelu(x):
    return 0.5 * x * (1.0 + lax.erf(x * (1.0 / math.sqrt(2.0))))


def _ln_qkv_body(x_ref, g_ref, b_ref, w_ref, bias_ref, q_ref, k_ref, v_ref):
    x = x_ref[...]
    m = jnp.mean(x, axis=-1, keepdims=True)
    c = x - m
    v = jnp.mean(c * c, axis=-1, keepdims=True)
    h = c / jnp.sqrt(v + 1e-5) * g_ref[...] + b_ref[...]
    qkv = _f32dot_nt(h, w_ref[...]) + bias_ref[...]
    for hh in range(_H):
        q_ref[hh] = qkv[:, hh * _HD:(hh + 1) * _HD]
        k_ref[hh] = qkv[:, _D + hh * _HD:_D + (hh + 1) * _HD]
        v_ref[hh] = qkv[:, 2 * _D + hh * _HD:2 * _D + (hh + 1) * _HD]


def _ln_qkv(x2d, g, b, w, bias):
    hs = jax.ShapeDtypeStruct((_H, _T, _HD), _F32)
    return pl.pallas_call(
        _ln_qkv_body,
        grid=(_N // _BR,),
        in_specs=[
            pl.BlockSpec((_BR, _D), lambda i: (i, 0)),
            pl.BlockSpec((1, _D), lambda i: (0, 0)),
            pl.BlockSpec((1, _D), lambda i: (0, 0)),
            pl.BlockSpec((3 * _D, _D), lambda i: (0, 0)),
            pl.BlockSpec((1, 3 * _D), lambda i: (0, 0)),
        ],
        out_specs=[pl.BlockSpec((_H, _BR, _HD), lambda i: (0, i, 0))] * 3,
        out_shape=[hs, hs, hs],
    )(x2d, g.reshape(1, _D), b.reshape(1, _D), w, bias.reshape(1, 3 * _D))


def _attn_body(q_ref, k_ref, v_ref, o_ref):
    qi = pl.program_id(1)
    q = q_ref[0]
    scale = 1.0 / math.sqrt(_HD)

    def tile(ki, carry, masked):
        m, l, acc = carry
        k = k_ref[0, pl.ds(ki * _BQ, _BQ), :]
        s = _f32dot_nt(q, k) * scale
        if masked:
            rows = lax.broadcasted_iota(jnp.int32, (_BQ, _BQ), 0)
            cols = lax.broadcasted_iota(jnp.int32, (_BQ, _BQ), 1)
            s = jnp.where(rows >= cols, s, -jnp.inf)
        mn = jnp.maximum(m, jnp.max(s, axis=1, keepdims=True))
        p = jnp.exp(s - mn)
        alpha = jnp.exp(m - mn)
        l2 = l * alpha + jnp.sum(p, axis=1, keepdims=True)
        v = v_ref[0, pl.ds(ki * _BQ, _BQ), :]
        acc2 = acc * alpha + _f32dot(p, v)
        return mn, l2, acc2

    m0 = jnp.full((_BQ, 1), -jnp.inf, _F32)
    l0 = jnp.zeros((_BQ, 1), _F32)
    a0 = jnp.zeros((_BQ, _HD), _F32)
    carry = lax.fori_loop(0, qi, lambda ki, c: tile(ki, c, False),
                          (m0, l0, a0))
    m, l, acc = tile(qi, carry, True)
    o_ref[0] = acc / l


def _attention(q, k, v):
    return pl.pallas_call(
        _attn_body,
        grid=(_H, _T // _BQ),
        in_specs=[
            pl.BlockSpec((1, _BQ, _HD), lambda h, i: (h, i, 0)),
            pl.BlockSpec((1, _T, _HD), lambda h, i: (h, 0, 0)),
            pl.BlockSpec((1, _T, _HD), lambda h, i: (h, 0, 0)),
        ],
        out_specs=pl.BlockSpec((1, _BQ, _HD), lambda h, i: (h, i, 0)),
        out_shape=jax.ShapeDtypeStruct((_H, _T, _HD), _F32),
    )(q, k, v)


def _proj_body(y_ref, x_ref, w_ref, b_ref, g2_ref, b2_ref, wg_ref,
               x1_ref, h2_ref, lg_ref):
    y = jnp.concatenate([y_ref[hh] for hh in range(_H)], axis=1)
    x1 = x_ref[...] + _f32dot_nt(y, w_ref[...]) + b_ref[...]
    x1_ref[...] = x1
    m = jnp.mean(x1, axis=-1, keepdims=True)
    c = x1 - m
    v = jnp.mean(c * c, axis=-1, keepdims=True)
    h2 = c / jnp.sqrt(v + 1e-5) * g2_ref[...] + b2_ref[...]
    h2_ref[...] = h2
    lg_ref[...] = _f32dot_nt(h2, wg_ref[...])


def _proj_ln2_logits(y, x2d, w, b, g2, b2, wg):
    return pl.pallas_call(
        _proj_body,
        grid=(_N // _BR,),
        in_specs=[
            pl.BlockSpec((_H, _BR, _HD), lambda i: (0, i, 0)),
            pl.BlockSpec((_BR, _D), lambda i: (i, 0)),
            pl.BlockSpec((_D, _D), lambda i: (0, 0)),
            pl.BlockSpec((1, _D), lambda i: (0, 0)),
            pl.BlockSpec((1, _D), lambda i: (0, 0)),
            pl.BlockSpec((1, _D), lambda i: (0, 0)),
            pl.BlockSpec((_E, _D), lambda i: (0, 0)),
        ],
        out_specs=[
            pl.BlockSpec((_BR, _D), lambda i: (i, 0)),
            pl.BlockSpec((_BR, _D), lambda i: (i, 0)),
            pl.BlockSpec((_BR, _E), lambda i: (i, 0)),
        ],
        out_shape=[
            jax.ShapeDtypeStruct((_N, _D), _F32),
            jax.ShapeDtypeStruct((_N, _D), _F32),
            jax.ShapeDtypeStruct((_N, _E), _F32),
        ],
    )(y, x2d, w, b.reshape(1, _D), g2.reshape(1, _D), b2.reshape(1, _D), wg)


_RCH = 256


def _router_body(nworkers, lg_ref, rt_ref, cnt_ref, oh0_ref, oh1_ref, c0_ref, c1_ref):
    l = lg_ref[...]
    eidx = lax.broadcasted_iota(jnp.int32, (_N, _E), 1)
    m0 = jnp.max(l, axis=1, keepdims=True)
    e0 = jnp.min(jnp.where(l == m0, eidx, _E), axis=1, keepdims=True)
    oh0 = (eidx == e0)
    lm = jnp.where(oh0, -jnp.inf, l)
    m1 = jnp.max(lm, axis=1, keepdims=True)
    e1 = jnp.min(jnp.where(lm == m1, eidx, _E), axis=1, keepdims=True)
    oh1 = (eidx == e1)
    t = jnp.exp(m1 - m0)
    w0 = 1.0 / (1.0 + t)
    w1 = t * w0
    oh0_ref[...] = oh0.astype(_F32)
    oh1_ref[...] = oh1.astype(_F32)

    tril = (lax.broadcasted_iota(jnp.int32, (_RCH, _RCH), 0)
            >= lax.broadcasted_iota(jnp.int32, (_RCH, _RCH), 1)).astype(_F32)

    def scan(oh_ref, cum_ref):
        def chunk(i, tot):
            oh = oh_ref[pl.ds(i * _RCH, _RCH), :]
            cum_ref[pl.ds(i * _RCH, _RCH), :] = _f32dot(tril, oh) + tot
            return tot + jnp.sum(oh, axis=0, keepdims=True)
        return lax.fori_loop(0, _N // _RCH, chunk, jnp.zeros((1, _E), _F32))

    tot0 = scan(oh0_ref, c0_ref)
    tot1 = scan(oh1_ref, c1_ref)
    cnt_ref[...] = jnp.minimum(tot0 + tot1, float(_CAP)).astype(jnp.int32)

    oh0f = oh0_ref[...]
    oh1f = oh1_ref[...]
    p0 = jnp.sum(oh0f * c0_ref[...], axis=1, keepdims=True) - 1.0
    p1 = jnp.sum(oh1f * (c1_ref[...] + tot0), axis=1, keepdims=True) - 1.0

    ppw = _NPAIR // nworkers
    nidx = lax.broadcasted_iota(jnp.int32, (_N, 1), 0)
    dump0 = (_SLOTS + nidx // ppw).astype(_F32)
    dump1 = (_SLOTS + (_N + nidx) // ppw).astype(_F32)
    keep0 = p0 < _CAP
    keep1 = p1 < _CAP
    e0f = e0.astype(_F32)
    e1f = e1.astype(_F32)
    d0 = jnp.where(keep0, e0f * _CAP + p0, dump0)
    d1 = jnp.where(keep1, e1f * _CAP + p1, dump1)
    w0 = jnp.where(keep0, w0, 0.0)
    w1 = jnp.where(keep1, w1, 0.0)
    z = jnp.zeros((_N, 4), _F32)
    rt_ref[...] = jnp.concatenate([d0, d1, w0, w1, z], axis=1)


def _router(logits, nworkers):
    return pl.pallas_call(
        functools.partial(_router_body, nworkers),
        grid=(1,),
        in_specs=[pl.BlockSpec((_N, _E), lambda i: (0, 0))],
        out_specs=[
            pl.BlockSpec((_N, 8), lambda i: (0, 0)),
            pl.BlockSpec((1, _E), lambda i: (0, 0)),
        ],
        out_shape=[
            jax.ShapeDtypeStruct((_N, 8), _F32),
            jax.ShapeDtypeStruct((1, _E), jnp.int32),
        ],
        scratch_shapes=[pltpu.VMEM((_N, _E), _F32) for _ in range(4)],
    )(logits)


def _sc_meshinfo():
    info = plsc.get_sparse_core_info()
    return info.num_cores, info.num_subcores


def _dispatch_sc(h2, slots):
    nc, ns = _sc_meshinfo()
    nw = nc * ns
    ppw = _NPAIR // nw
    mesh = plsc.VectorSubcoreMesh(core_axis_name="c", subcore_axis_name="s")

    @functools.partial(
        pl.kernel, mesh=mesh,
        out_type=jax.ShapeDtypeStruct((_SLOTS + nw, _D), _F32),
        scratch_types=[
            pltpu.VMEM((ppw,), jnp.int32),
            pltpu.VMEM((ppw, _D), _F32),
        ],
    )
    def k(h2_hbm, slots_hbm, out_hbm, idx_v, rows_v):
        wid = lax.axis_index("s") * nc + lax.axis_index("c")
        tok = (wid * ppw) % _N
        pltpu.sync_copy(slots_hbm.at[wid], idx_v)
        pltpu.sync_copy(h2_hbm.at[pl.ds(tok, ppw)], rows_v)
        pltpu.sync_copy(rows_v, out_hbm.at[idx_v])

    return k(h2, slots)


def _gather_sc(o_flat, slots):
    nc, ns = _sc_meshinfo()
    nw = nc * ns
    ppw = _NPAIR // nw
    mesh = plsc.VectorSubcoreMesh(core_axis_name="c", subcore_axis_name="s")

    @functools.partial(
        pl.kernel, mesh=mesh,
        out_type=jax.ShapeDtypeStruct((_NPAIR, _D), _F32),
        scratch_types=[
            pltpu.VMEM((ppw,), jnp.int32),
            pltpu.VMEM((ppw, _D), _F32),
            pltpu.SemaphoreType.DMA,
        ],
    )
    def k(o_hbm, slots_hbm, out_hbm, idx_v, rows_v, sem):
        wid = lax.axis_index("s") * nc + lax.axis_index("c")
        pltpu.sync_copy(slots_hbm.at[wid], idx_v)
        pltpu.async_copy(o_hbm.at[idx_v], rows_v, sem).wait()
        pltpu.sync_copy(rows_v, out_hbm.at[pl.ds(wid * ppw, ppw)])

    return k(o_flat, slots)


def _mlp_body(disp_ref, wfc_ref, bfc_ref, wpr_ref, bpr_ref, o_ref, acc_ref):
    j = pl.program_id(1)
    a = _gelu(_f32dot(disp_ref[...], wfc_ref[0]) + bfc_ref[0])
    part = _f32dot(a, wpr_ref[0])

    @pl.when(j == 0)
    def _():
        acc_ref[...] = part

    @pl.when(j > 0)
    def _():
        acc_ref[...] += part

    @pl.when(j == _DFF // _D - 1)
    def _():
        o_ref[...] = acc_ref[...] + bpr_ref[0]


def _expert_mlps(disp, c_fc, fc_bias, c_proj_e, proj_bias, nrows):
    nj = _DFF // _D
    return pl.pallas_call(
        _mlp_body,
        grid=(_E, nj),
        in_specs=[
            pl.BlockSpec((_CAP, _D), lambda e, j: (e, 0)),
            pl.BlockSpec((1, _D, _D), lambda e, j: (e, 0, j)),
            pl.BlockSpec((1, 1, _D), lambda e, j: (e, 0, j)),
            pl.BlockSpec((1, _D, _D), lambda e, j: (e, j, 0)),
            pl.BlockSpec((1, 1, _D), lambda e, j: (e, 0, 0)),
        ],
        out_specs=pl.BlockSpec((_CAP, _D), lambda e, j: (e, 0)),
        out_shape=jax.ShapeDtypeStruct((nrows, _D), _F32),
        scratch_shapes=[pltpu.VMEM((_CAP, _D), _F32)],
    )(disp, c_fc, fc_bias, c_proj_e, proj_bias)


def _combine_body(x1_ref, g0_ref, g1_ref, w0_ref, w1_ref, o_ref):
    w0 = w0_ref[...]
    w1 = w1_ref[...]
    o_ref[...] = (x1_ref[...]
                  + jnp.where(w0 == 0.0, 0.0, w0 * g0_ref[...])
                  + jnp.where(w1 == 0.0, 0.0, w1 * g1_ref[...]))


def _combine(x1, g, w0, w1):
    nb = _N // _BR
    return pl.pallas_call(
        _combine_body,
        grid=(nb,),
        in_specs=[
            pl.BlockSpec((_BR, _D), lambda i: (i, 0)),
            pl.BlockSpec((_BR, _D), lambda i: (i, 0)),
            pl.BlockSpec((_BR, _D), lambda i, _nb=nb: (i + _nb, 0)),
            pl.BlockSpec((_BR, 1), lambda i: (i, 0)),
            pl.BlockSpec((_BR, 1), lambda i: (i, 0)),
        ],
        out_specs=pl.BlockSpec((_BR, _D), lambda i: (i, 0)),
        out_shape=jax.ShapeDtypeStruct((_N, _D), _F32),
    )(x1, g, g, w0, w1)


def kernel(x, ln1_g, ln1_b, c_attn_w, c_attn_b, c_proj_w, c_proj_b,
           ln2_g, ln2_b, w_g, c_fc, fc_bias, c_proj_e, proj_bias):
    nc, ns = _sc_meshinfo()
    nw = nc * ns

    x2d = x.reshape(_N, _D)
    q, k, v = _ln_qkv(x2d, ln1_g, ln1_b, c_attn_w, c_attn_b)
    y = _attention(q, k, v)

    x1, h2, logits = _proj_ln2_logits(y, x2d, c_proj_w, c_proj_b,
                                      ln2_g, ln2_b, w_g)

    rt, cnt = _router(logits, nw)
    slots = jnp.concatenate([rt[:, 0], rt[:, 1]]).astype(jnp.int32)
    slots = slots.reshape(nw, _NPAIR // nw)
    w0 = rt[:, 2:3]
    w1 = rt[:, 3:4]

    disp = _dispatch_sc(h2, slots)
    o_flat = _expert_mlps(disp, c_fc, fc_bias, c_proj_e, proj_bias,
                          _SLOTS + nw)
    g = _gather_sc(o_flat, slots)
    out = _combine(x1, g, w0, w1)
    return out.reshape(_B, _T, _D)


# speedup vs baseline: 1.2463x; 1.0236x over previous
import functools
import math

import jax
import jax.numpy as jnp
from jax import lax
from jax.experimental import pallas as pl
from jax.experimental.pallas import tpu as pltpu
from jax.experimental.pallas import tpu_sc as plsc

_B, _T, _D, _H, _E, _K = 1, 2048, 768, 12, 8, 2
_DFF = 4 * _D
_N = _B * _T
_cc = math.floor(_K * 1.25 * _N / _E)
_cc += _cc % 2
_CAP = max(_cc, 4)
_HD = _D // _H
_SLOTS = _E * _CAP
_NPAIR = _K * _N

_BQ = 1024
_BR = 512
_F32 = jnp.float32


def _f32dot(a, b):
    return jnp.dot(a, b, preferred_element_type=_F32)


def _f32dot_nt(a, b):
    return lax.dot_general(a, b, (((1,), (1,)), ((), ())),
                           preferred_element_type=_F32)


def _gelu(x):
    return 0.5 * x * (1.0 + lax.erf(x * (1.0 / math.sqrt(2.0))))


def _ln_qkv_body(x_ref, g_ref, b_ref, w_ref, bias_ref, q_ref, k_ref, v_ref):
    x = x_ref[...]
    m = jnp.mean(x, axis=-1, keepdims=True)
    c = x - m
    v = jnp.mean(c * c, axis=-1, keepdims=True)
    h = c / jnp.sqrt(v + 1e-5) * g_ref[...] + b_ref[...]
    qkv = _f32dot_nt(h, w_ref[...]) + bias_ref[...]
    for hh in range(_H):
        q_ref[hh] = qkv[:, hh * _HD:(hh + 1) * _HD]
        k_ref[hh] = qkv[:, _D + hh * _HD:_D + (hh + 1) * _HD]
        v_ref[hh] = qkv[:, 2 * _D + hh * _HD:2 * _D + (hh + 1) * _HD]


def _ln_qkv(x2d, g, b, w, bias):
    hs = jax.ShapeDtypeStruct((_H, _T, _HD), _F32)
    return pl.pallas_call(
        _ln_qkv_body,
        grid=(_N // _BR,),
        in_specs=[
            pl.BlockSpec((_BR, _D), lambda i: (i, 0)),
            pl.BlockSpec((1, _D), lambda i: (0, 0)),
            pl.BlockSpec((1, _D), lambda i: (0, 0)),
            pl.BlockSpec((3 * _D, _D), lambda i: (0, 0)),
            pl.BlockSpec((1, 3 * _D), lambda i: (0, 0)),
        ],
        out_specs=[pl.BlockSpec((_H, _BR, _HD), lambda i: (0, i, 0))] * 3,
        out_shape=[hs, hs, hs],
    )(x2d, g.reshape(1, _D), b.reshape(1, _D), w, bias.reshape(1, 3 * _D))


def _attn_body(q_ref, k_ref, v_ref, o_ref):
    qi = pl.program_id(1)
    q = q_ref[0]
    scale = 1.0 / math.sqrt(_HD)

    def tile(ki, carry, masked):
        m, l, acc = carry
        k = k_ref[0, pl.ds(ki * _BQ, _BQ), :]
        s = _f32dot_nt(q, k) * scale
        if masked:
            rows = lax.broadcasted_iota(jnp.int32, (_BQ, _BQ), 0)
            cols = lax.broadcasted_iota(jnp.int32, (_BQ, _BQ), 1)
            s = jnp.where(rows >= cols, s, -jnp.inf)
        mn = jnp.maximum(m, jnp.max(s, axis=1, keepdims=True))
        p = jnp.exp(s - mn)
        alpha = jnp.exp(m - mn)
        l2 = l * alpha + jnp.sum(p, axis=1, keepdims=True)
        v = v_ref[0, pl.ds(ki * _BQ, _BQ), :]
        acc2 = acc * alpha + _f32dot(p, v)
        return mn, l2, acc2

    m0 = jnp.full((_BQ, 1), -jnp.inf, _F32)
    l0 = jnp.zeros((_BQ, 1), _F32)
    a0 = jnp.zeros((_BQ, _HD), _F32)
    carry = lax.fori_loop(0, qi, lambda ki, c: tile(ki, c, False),
                          (m0, l0, a0))
    m, l, acc = tile(qi, carry, True)
    o_ref[0] = acc / l


def _attention(q, k, v):
    return pl.pallas_call(
        _attn_body,
        grid=(_H, _T // _BQ),
        in_specs=[
            pl.BlockSpec((1, _BQ, _HD), lambda h, i: (h, i, 0)),
            pl.BlockSpec((1, _T, _HD), lambda h, i: (h, 0, 0)),
            pl.BlockSpec((1, _T, _HD), lambda h, i: (h, 0, 0)),
        ],
        out_specs=pl.BlockSpec((1, _BQ, _HD), lambda h, i: (h, i, 0)),
        out_shape=jax.ShapeDtypeStruct((_H, _T, _HD), _F32),
    )(q, k, v)


def _proj_body(y_ref, x_ref, w_ref, b_ref, g2_ref, b2_ref, wg_ref,
               x1_ref, h2_ref, lg_ref):
    y = jnp.concatenate([y_ref[hh] for hh in range(_H)], axis=1)
    x1 = x_ref[...] + _f32dot_nt(y, w_ref[...]) + b_ref[...]
    x1_ref[...] = x1
    m = jnp.mean(x1, axis=-1, keepdims=True)
    c = x1 - m
    v = jnp.mean(c * c, axis=-1, keepdims=True)
    h2 = c / jnp.sqrt(v + 1e-5) * g2_ref[...] + b2_ref[...]
    h2_ref[...] = h2
    lg_ref[...] = _f32dot_nt(h2, wg_ref[...])


def _proj_ln2_logits(y, x2d, w, b, g2, b2, wg):
    return pl.pallas_call(
        _proj_body,
        grid=(_N // _BR,),
        in_specs=[
            pl.BlockSpec((_H, _BR, _HD), lambda i: (0, i, 0)),
            pl.BlockSpec((_BR, _D), lambda i: (i, 0)),
            pl.BlockSpec((_D, _D), lambda i: (0, 0)),
            pl.BlockSpec((1, _D), lambda i: (0, 0)),
            pl.BlockSpec((1, _D), lambda i: (0, 0)),
            pl.BlockSpec((1, _D), lambda i: (0, 0)),
            pl.BlockSpec((_E, _D), lambda i: (0, 0)),
        ],
        out_specs=[
            pl.BlockSpec((_BR, _D), lambda i: (i, 0)),
            pl.BlockSpec((_BR, _D), lambda i: (i, 0)),
            pl.BlockSpec((_BR, _E), lambda i: (i, 0)),
        ],
        out_shape=[
            jax.ShapeDtypeStruct((_N, _D), _F32),
            jax.ShapeDtypeStruct((_N, _D), _F32),
            jax.ShapeDtypeStruct((_N, _E), _F32),
        ],
    )(y, x2d, w, b.reshape(1, _D), g2.reshape(1, _D), b2.reshape(1, _D), wg)


_RCH = 256


def _router_body(nworkers, lg_ref, rt_ref, cnt_ref, oh0_ref, oh1_ref, c0_ref, c1_ref):
    l = lg_ref[...]
    eidx = lax.broadcasted_iota(jnp.int32, (_N, _E), 1)
    m0 = jnp.max(l, axis=1, keepdims=True)
    e0 = jnp.min(jnp.where(l == m0, eidx, _E), axis=1, keepdims=True)
    oh0 = (eidx == e0)
    lm = jnp.where(oh0, -jnp.inf, l)
    m1 = jnp.max(lm, axis=1, keepdims=True)
    e1 = jnp.min(jnp.where(lm == m1, eidx, _E), axis=1, keepdims=True)
    oh1 = (eidx == e1)
    t = jnp.exp(m1 - m0)
    w0 = 1.0 / (1.0 + t)
    w1 = t * w0
    oh0_ref[...] = oh0.astype(_F32)
    oh1_ref[...] = oh1.astype(_F32)

    tril = (lax.broadcasted_iota(jnp.int32, (_RCH, _RCH), 0)
            >= lax.broadcasted_iota(jnp.int32, (_RCH, _RCH), 1)).astype(_F32)

    def scan(oh_ref, cum_ref):
        def chunk(i, tot):
            oh = oh_ref[pl.ds(i * _RCH, _RCH), :]
            cum_ref[pl.ds(i * _RCH, _RCH), :] = _f32dot(tril, oh) + tot
            return tot + jnp.sum(oh, axis=0, keepdims=True)
        return lax.fori_loop(0, _N // _RCH, chunk, jnp.zeros((1, _E), _F32))

    tot0 = scan(oh0_ref, c0_ref)
    tot1 = scan(oh1_ref, c1_ref)
    cnt_ref[...] = jnp.minimum(tot0 + tot1, float(_CAP)).astype(jnp.int32)

    oh0f = oh0_ref[...]
    oh1f = oh1_ref[...]
    p0 = jnp.sum(oh0f * c0_ref[...], axis=1, keepdims=True) - 1.0
    p1 = jnp.sum(oh1f * (c1_ref[...] + tot0), axis=1, keepdims=True) - 1.0

    ppw = _NPAIR // nworkers
    nidx = lax.broadcasted_iota(jnp.int32, (_N, 1), 0)
    dump0 = (_SLOTS + nidx // ppw).astype(_F32)
    dump1 = (_SLOTS + (_N + nidx) // ppw).astype(_F32)
    keep0 = p0 < _CAP
    keep1 = p1 < _CAP
    e0f = e0.astype(_F32)
    e1f = e1.astype(_F32)
    d0 = jnp.where(keep0, e0f * _CAP + p0, dump0)
    d1 = jnp.where(keep1, e1f * _CAP + p1, dump1)
    w0 = jnp.where(keep0, w0, 0.0)
    w1 = jnp.where(keep1, w1, 0.0)
    z = jnp.zeros((_N, 4), _F32)
    rt_ref[...] = jnp.concatenate([d0, d1, w0, w1, z], axis=1)


def _router(logits, nworkers):
    return pl.pallas_call(
        functools.partial(_router_body, nworkers),
        grid=(1,),
        in_specs=[pl.BlockSpec((_N, _E), lambda i: (0, 0))],
        out_specs=[
            pl.BlockSpec((_N, 8), lambda i: (0, 0)),
            pl.BlockSpec((1, _E), lambda i: (0, 0)),
        ],
        out_shape=[
            jax.ShapeDtypeStruct((_N, 8), _F32),
            jax.ShapeDtypeStruct((1, _E), jnp.int32),
        ],
        scratch_shapes=[pltpu.VMEM((_N, _E), _F32) for _ in range(4)],
    )(logits)


def _sc_meshinfo():
    info = plsc.get_sparse_core_info()
    return info.num_cores, info.num_subcores


def _dispatch_sc(h2, slots):
    nc, ns = _sc_meshinfo()
    nw = nc * ns
    ppw = _NPAIR // nw
    mesh = plsc.VectorSubcoreMesh(core_axis_name="c", subcore_axis_name="s")

    @functools.partial(
        pl.kernel, mesh=mesh,
        out_type=jax.ShapeDtypeStruct((_SLOTS + nw, _D), _F32),
        scratch_types=[
            pltpu.VMEM((ppw,), jnp.int32),
            pltpu.VMEM((ppw, _D), _F32),
        ],
    )
    def k(h2_hbm, slots_hbm, out_hbm, idx_v, rows_v):
        wid = lax.axis_index("s") * nc + lax.axis_index("c")
        tok = (wid * ppw) % _N
        pltpu.sync_copy(slots_hbm.at[wid], idx_v)
        pltpu.sync_copy(h2_hbm.at[pl.ds(tok, ppw)], rows_v)
        pltpu.sync_copy(rows_v, out_hbm.at[idx_v])

    return k(h2, slots)


def _gather_sc(o_flat, slots):
    nc, ns = _sc_meshinfo()
    nw = nc * ns
    ppw = _NPAIR // nw
    mesh = plsc.VectorSubcoreMesh(core_axis_name="c", subcore_axis_name="s")

    @functools.partial(
        pl.kernel, mesh=mesh,
        out_type=jax.ShapeDtypeStruct((_NPAIR, _D), _F32),
        scratch_types=[
            pltpu.VMEM((ppw,), jnp.int32),
            pltpu.VMEM((ppw, _D), _F32),
            pltpu.SemaphoreType.DMA,
        ],
    )
    def k(o_hbm, slots_hbm, out_hbm, idx_v, rows_v, sem):
        wid = lax.axis_index("s") * nc + lax.axis_index("c")
        pltpu.sync_copy(slots_hbm.at[wid], idx_v)
        pltpu.async_copy(o_hbm.at[idx_v], rows_v, sem).wait()
        pltpu.sync_copy(rows_v, out_hbm.at[pl.ds(wid * ppw, ppw)])

    return k(o_flat, slots)


def _mlp_body(disp_ref, wfc_ref, bfc_ref, wpr_ref, bpr_ref, o_ref, acc_ref):
    j = pl.program_id(1)
    a = _gelu(_f32dot(disp_ref[...], wfc_ref[0]) + bfc_ref[0])
    part = _f32dot(a, wpr_ref[0])

    @pl.when(j == 0)
    def _():
        acc_ref[...] = part

    @pl.when(j > 0)
    def _():
        acc_ref[...] += part

    @pl.when(j == _DFF // _D - 1)
    def _():
        o_ref[...] = acc_ref[...] + bpr_ref[0]


def _expert_mlps(disp, c_fc, fc_bias, c_proj_e, proj_bias, nrows):
    nj = _DFF // _D
    return pl.pallas_call(
        _mlp_body,
        grid=(_E, nj),
        in_specs=[
            pl.BlockSpec((_CAP, _D), lambda e, j: (e, 0)),
            pl.BlockSpec((1, _D, _D), lambda e, j: (e, 0, j)),
            pl.BlockSpec((1, 1, _D), lambda e, j: (e, 0, j)),
            pl.BlockSpec((1, _D, _D), lambda e, j: (e, j, 0)),
            pl.BlockSpec((1, 1, _D), lambda e, j: (e, 0, 0)),
        ],
        out_specs=pl.BlockSpec((_CAP, _D), lambda e, j: (e, 0)),
        out_shape=jax.ShapeDtypeStruct((nrows, _D), _F32),
        scratch_shapes=[pltpu.VMEM((_CAP, _D), _F32)],
    )(disp, c_fc, fc_bias, c_proj_e, proj_bias)


def _combine_body(x1_ref, g0_ref, g1_ref, w0_ref, w1_ref, o_ref):
    w0 = w0_ref[...]
    w1 = w1_ref[...]
    o_ref[...] = (x1_ref[...]
                  + jnp.where(w0 == 0.0, 0.0, w0 * g0_ref[...])
                  + jnp.where(w1 == 0.0, 0.0, w1 * g1_ref[...]))


def _combine(x1, g, w0, w1):
    nb = _N // _BR
    return pl.pallas_call(
        _combine_body,
        grid=(nb,),
        in_specs=[
            pl.BlockSpec((_BR, _D), lambda i: (i, 0)),
            pl.BlockSpec((_BR, _D), lambda i: (i, 0)),
            pl.BlockSpec((_BR, _D), lambda i, _nb=nb: (i + _nb, 0)),
            pl.BlockSpec((_BR, 1), lambda i: (i, 0)),
            pl.BlockSpec((_BR, 1), lambda i: (i, 0)),
        ],
        out_specs=pl.BlockSpec((_BR, _D), lambda i: (i, 0)),
        out_shape=jax.ShapeDtypeStruct((_N, _D), _F32),
    )(x1, g, g, w0, w1)


def kernel(x, ln1_g, ln1_b, c_attn_w, c_attn_b, c_proj_w, c_proj_b,
           ln2_g, ln2_b, w_g, c_fc, fc_bias, c_proj_e, proj_bias):
    nc, ns = _sc_meshinfo()
    nw = nc * ns

    x2d = x.reshape(_N, _D)
    q, k, v = _ln_qkv(x2d, ln1_g, ln1_b, c_attn_w, c_attn_b)
    y = _attention(q, k, v)

    x1, h2, logits = _proj_ln2_logits(y, x2d, c_proj_w, c_proj_b,
                                      ln2_g, ln2_b, w_g)

    rt, cnt = _router(logits, nw)
    slots = jnp.concatenate([rt[:, 0], rt[:, 1]]).astype(jnp.int32)
    slots = slots.reshape(nw, _NPAIR // nw)
    w0 = rt[:, 2:3]
    w1 = rt[:, 3:4]

    disp = _dispatch_sc(h2, slots)
    o_flat = _expert_mlps(disp, c_fc, fc_bias, c_proj_e, proj_bias,
                          _SLOTS + nw)
    g = _gather_sc(o_flat, slots)
    out = _combine(x1, g, w0, w1)
    return out.reshape(_B, _T, _D)


# speedup vs baseline: 1.3375x; 1.0732x over previous
import functools
import math

import jax
import jax.numpy as jnp
from jax import lax
from jax.experimental import pallas as pl
from jax.experimental.pallas import tpu as pltpu
from jax.experimental.pallas import tpu_sc as plsc

_B, _T, _D, _H, _E, _K = 1, 2048, 768, 12, 8, 2
_DFF = 4 * _D
_N = _B * _T
_cc = math.floor(_K * 1.25 * _N / _E)
_cc += _cc % 2
_CAP = max(_cc, 4)
_HD = _D // _H
_SLOTS = _E * _CAP
_NPAIR = _K * _N

_BQ = 1024
_BR = 512
_F32 = jnp.float32


def _f32dot(a, b):
    return jnp.dot(a, b, preferred_element_type=_F32)


def _f32dot_nt(a, b):
    return lax.dot_general(a, b, (((1,), (1,)), ((), ())),
                           preferred_element_type=_F32)


def _gelu(x):
    return 0.5 * x * (1.0 + lax.erf(x * (1.0 / math.sqrt(2.0))))


def _ln_qkv_body(x_ref, g_ref, b_ref, w_ref, bias_ref, q_ref, k_ref, v_ref):
    x = x_ref[...]
    m = jnp.mean(x, axis=-1, keepdims=True)
    c = x - m
    v = jnp.mean(c * c, axis=-1, keepdims=True)
    h = c / jnp.sqrt(v + 1e-5) * g_ref[...] + b_ref[...]
    qkv = _f32dot_nt(h, w_ref[...]) + bias_ref[...]
    for hh in range(_H):
        q_ref[hh] = qkv[:, hh * _HD:(hh + 1) * _HD]
        k_ref[hh] = qkv[:, _D + hh * _HD:_D + (hh + 1) * _HD]
        v_ref[hh] = qkv[:, 2 * _D + hh * _HD:2 * _D + (hh + 1) * _HD]


def _ln_qkv(x2d, g, b, w, bias):
    hs = jax.ShapeDtypeStruct((_H, _T, _HD), _F32)
    return pl.pallas_call(
        _ln_qkv_body,
        grid=(_N // _BR,),
        in_specs=[
            pl.BlockSpec((_BR, _D), lambda i: (i, 0)),
            pl.BlockSpec((1, _D), lambda i: (0, 0)),
            pl.BlockSpec((1, _D), lambda i: (0, 0)),
            pl.BlockSpec((3 * _D, _D), lambda i: (0, 0)),
            pl.BlockSpec((1, 3 * _D), lambda i: (0, 0)),
        ],
        out_specs=[pl.BlockSpec((_H, _BR, _HD), lambda i: (0, i, 0))] * 3,
        out_shape=[hs, hs, hs],
    )(x2d, g.reshape(1, _D), b.reshape(1, _D), w, bias.reshape(1, 3 * _D))


def _attn_body(q_ref, k_ref, v_ref, o_ref):
    qi = pl.program_id(1)
    q = q_ref[0]
    scale = 1.0 / math.sqrt(_HD)

    def tile(ki, carry, masked):
        m, l, acc = carry
        k = k_ref[0, pl.ds(ki * _BQ, _BQ), :]
        s = _f32dot_nt(q, k) * scale
        if masked:
            rows = lax.broadcasted_iota(jnp.int32, (_BQ, _BQ), 0)
            cols = lax.broadcasted_iota(jnp.int32, (_BQ, _BQ), 1)
            s = jnp.where(rows >= cols, s, -jnp.inf)
        mn = jnp.maximum(m, jnp.max(s, axis=1, keepdims=True))
        p = jnp.exp(s - mn)
        alpha = jnp.exp(m - mn)
        l2 = l * alpha + jnp.sum(p, axis=1, keepdims=True)
        v = v_ref[0, pl.ds(ki * _BQ, _BQ), :]
        acc2 = acc * alpha + _f32dot(p, v)
        return mn, l2, acc2

    m0 = jnp.full((_BQ, 1), -jnp.inf, _F32)
    l0 = jnp.zeros((_BQ, 1), _F32)
    a0 = jnp.zeros((_BQ, _HD), _F32)
    carry = lax.fori_loop(0, qi, lambda ki, c: tile(ki, c, False),
                          (m0, l0, a0))
    m, l, acc = tile(qi, carry, True)
    o_ref[0] = acc / l


def _attention(q, k, v):
    return pl.pallas_call(
        _attn_body,
        grid=(_H, _T // _BQ),
        in_specs=[
            pl.BlockSpec((1, _BQ, _HD), lambda h, i: (h, i, 0)),
            pl.BlockSpec((1, _T, _HD), lambda h, i: (h, 0, 0)),
            pl.BlockSpec((1, _T, _HD), lambda h, i: (h, 0, 0)),
        ],
        out_specs=pl.BlockSpec((1, _BQ, _HD), lambda h, i: (h, i, 0)),
        out_shape=jax.ShapeDtypeStruct((_H, _T, _HD), _F32),
    )(q, k, v)


def _proj_body(y_ref, x_ref, w_ref, b_ref, g2_ref, b2_ref, wg_ref,
               x1_ref, h2_ref, lg_ref):
    y = jnp.concatenate([y_ref[hh] for hh in range(_H)], axis=1)
    x1 = x_ref[...] + _f32dot_nt(y, w_ref[...]) + b_ref[...]
    x1_ref[...] = x1
    m = jnp.mean(x1, axis=-1, keepdims=True)
    c = x1 - m
    v = jnp.mean(c * c, axis=-1, keepdims=True)
    h2 = c / jnp.sqrt(v + 1e-5) * g2_ref[...] + b2_ref[...]
    h2_ref[...] = h2
    lg_ref[...] = _f32dot_nt(h2, wg_ref[...])


def _proj_ln2_logits(y, x2d, w, b, g2, b2, wg):
    return pl.pallas_call(
        _proj_body,
        grid=(_N // _BR,),
        in_specs=[
            pl.BlockSpec((_H, _BR, _HD), lambda i: (0, i, 0)),
            pl.BlockSpec((_BR, _D), lambda i: (i, 0)),
            pl.BlockSpec((_D, _D), lambda i: (0, 0)),
            pl.BlockSpec((1, _D), lambda i: (0, 0)),
            pl.BlockSpec((1, _D), lambda i: (0, 0)),
            pl.BlockSpec((1, _D), lambda i: (0, 0)),
            pl.BlockSpec((_E, _D), lambda i: (0, 0)),
        ],
        out_specs=[
            pl.BlockSpec((_BR, _D), lambda i: (i, 0)),
            pl.BlockSpec((_BR, _D), lambda i: (i, 0)),
            pl.BlockSpec((_BR, _E), lambda i: (i, 0)),
        ],
        out_shape=[
            jax.ShapeDtypeStruct((_N, _D), _F32),
            jax.ShapeDtypeStruct((_N, _D), _F32),
            jax.ShapeDtypeStruct((_N, _E), _F32),
        ],
    )(y, x2d, w, b.reshape(1, _D), g2.reshape(1, _D), b2.reshape(1, _D), wg)


_RCH = 256


def _router_body(nworkers, lg_ref, rt_ref, cnt_ref, oh0_ref, oh1_ref, c0_ref, c1_ref):
    l = lg_ref[...]
    eidx = lax.broadcasted_iota(jnp.int32, (_N, _E), 1)
    m0 = jnp.max(l, axis=1, keepdims=True)
    e0 = jnp.min(jnp.where(l == m0, eidx, _E), axis=1, keepdims=True)
    oh0 = (eidx == e0)
    lm = jnp.where(oh0, -jnp.inf, l)
    m1 = jnp.max(lm, axis=1, keepdims=True)
    e1 = jnp.min(jnp.where(lm == m1, eidx, _E), axis=1, keepdims=True)
    oh1 = (eidx == e1)
    t = jnp.exp(m1 - m0)
    w0 = 1.0 / (1.0 + t)
    w1 = t * w0
    oh0_ref[...] = oh0.astype(_F32)
    oh1_ref[...] = oh1.astype(_F32)

    tril = (lax.broadcasted_iota(jnp.int32, (_RCH, _RCH), 0)
            >= lax.broadcasted_iota(jnp.int32, (_RCH, _RCH), 1)).astype(_F32)

    def scan(oh_ref, cum_ref):
        def chunk(i, tot):
            oh = oh_ref[pl.ds(i * _RCH, _RCH), :]
            cum_ref[pl.ds(i * _RCH, _RCH), :] = _f32dot(tril, oh) + tot
            return tot + jnp.sum(oh, axis=0, keepdims=True)
        return lax.fori_loop(0, _N // _RCH, chunk, jnp.zeros((1, _E), _F32))

    tot0 = scan(oh0_ref, c0_ref)
    tot1 = scan(oh1_ref, c1_ref)
    cnt_ref[...] = jnp.minimum(tot0 + tot1, float(_CAP)).astype(jnp.int32)

    oh0f = oh0_ref[...]
    oh1f = oh1_ref[...]
    p0 = jnp.sum(oh0f * c0_ref[...], axis=1, keepdims=True) - 1.0
    p1 = jnp.sum(oh1f * (c1_ref[...] + tot0), axis=1, keepdims=True) - 1.0

    ppw = _NPAIR // nworkers
    nidx = lax.broadcasted_iota(jnp.int32, (_N, 1), 0)
    dump0 = (_SLOTS + nidx // ppw).astype(_F32)
    dump1 = (_SLOTS + (_N + nidx) // ppw).astype(_F32)
    keep0 = p0 < _CAP
    keep1 = p1 < _CAP
    e0f = e0.astype(_F32)
    e1f = e1.astype(_F32)
    d0 = jnp.where(keep0, e0f * _CAP + p0, dump0)
    d1 = jnp.where(keep1, e1f * _CAP + p1, dump1)
    w0 = jnp.where(keep0, w0, 0.0)
    w1 = jnp.where(keep1, w1, 0.0)
    z = jnp.zeros((_N, 4), _F32)
    rt_ref[...] = jnp.concatenate([d0, d1, w0, w1, z], axis=1)


def _router(logits, nworkers):
    return pl.pallas_call(
        functools.partial(_router_body, nworkers),
        grid=(1,),
        in_specs=[pl.BlockSpec((_N, _E), lambda i: (0, 0))],
        out_specs=[
            pl.BlockSpec((_N, 8), lambda i: (0, 0)),
            pl.BlockSpec((1, _E), lambda i: (0, 0)),
        ],
        out_shape=[
            jax.ShapeDtypeStruct((_N, 8), _F32),
            jax.ShapeDtypeStruct((1, _E), jnp.int32),
        ],
        scratch_shapes=[pltpu.VMEM((_N, _E), _F32) for _ in range(4)],
    )(logits)


def _sc_meshinfo():
    info = plsc.get_sparse_core_info()
    return info.num_cores, info.num_subcores


def _dispatch_sc(h2, slots):
    nc, ns = _sc_meshinfo()
    nw = nc * ns
    ppw = _NPAIR // nw
    mesh = plsc.VectorSubcoreMesh(core_axis_name="c", subcore_axis_name="s")

    @functools.partial(
        pl.kernel, mesh=mesh,
        out_type=jax.ShapeDtypeStruct((_SLOTS + nw, _D), _F32),
        scratch_types=[
            pltpu.VMEM((ppw,), jnp.int32),
            pltpu.VMEM((ppw, _D), _F32),
        ],
    )
    def k(h2_hbm, slots_hbm, out_hbm, idx_v, rows_v):
        wid = lax.axis_index("s") * nc + lax.axis_index("c")
        tok = (wid * ppw) % _N
        pltpu.sync_copy(slots_hbm.at[wid], idx_v)
        pltpu.sync_copy(h2_hbm.at[pl.ds(tok, ppw)], rows_v)
        pltpu.sync_copy(rows_v, out_hbm.at[idx_v])

    return k(h2, slots)


def _gather_sc(o_flat, slots):
    nc, ns = _sc_meshinfo()
    nw = nc * ns
    ppw = _NPAIR // nw
    mesh = plsc.VectorSubcoreMesh(core_axis_name="c", subcore_axis_name="s")

    @functools.partial(
        pl.kernel, mesh=mesh,
        out_type=jax.ShapeDtypeStruct((_NPAIR, _D), _F32),
        scratch_types=[
            pltpu.VMEM((ppw,), jnp.int32),
            pltpu.VMEM((ppw, _D), _F32),
            pltpu.SemaphoreType.DMA,
        ],
    )
    def k(o_hbm, slots_hbm, out_hbm, idx_v, rows_v, sem):
        wid = lax.axis_index("s") * nc + lax.axis_index("c")
        pltpu.sync_copy(slots_hbm.at[wid], idx_v)
        pltpu.async_copy(o_hbm.at[idx_v], rows_v, sem).wait()
        pltpu.sync_copy(rows_v, out_hbm.at[pl.ds(wid * ppw, ppw)])

    return k(o_flat, slots)


def _mlp_body(disp_ref, wfc_ref, bfc_ref, wpr_ref, bpr_ref, o_ref):
    a = _gelu(_f32dot(disp_ref[...], wfc_ref[0]) + bfc_ref[0])
    o_ref[...] = _f32dot(a, wpr_ref[0]) + bpr_ref[0]


def _expert_mlps(disp, c_fc, fc_bias, c_proj_e, proj_bias, nrows):
    return pl.pallas_call(
        _mlp_body,
        grid=(_E,),
        in_specs=[
            pl.BlockSpec((_CAP, _D), lambda e: (e, 0)),
            pl.BlockSpec((1, _D, _DFF), lambda e: (e, 0, 0)),
            pl.BlockSpec((1, 1, _DFF), lambda e: (e, 0, 0)),
            pl.BlockSpec((1, _DFF, _D), lambda e: (e, 0, 0)),
            pl.BlockSpec((1, 1, _D), lambda e: (e, 0, 0)),
        ],
        out_specs=pl.BlockSpec((_CAP, _D), lambda e: (e, 0)),
        out_shape=jax.ShapeDtypeStruct((nrows, _D), _F32),
    )(disp, c_fc, fc_bias, c_proj_e, proj_bias)


def _combine_body(x1_ref, g0_ref, g1_ref, w0_ref, w1_ref, o_ref):
    w0 = w0_ref[...]
    w1 = w1_ref[...]
    o_ref[...] = (x1_ref[...]
                  + jnp.where(w0 == 0.0, 0.0, w0 * g0_ref[...])
                  + jnp.where(w1 == 0.0, 0.0, w1 * g1_ref[...]))


def _combine(x1, g, w0, w1):
    nb = _N // _BR
    return pl.pallas_call(
        _combine_body,
        grid=(nb,),
        in_specs=[
            pl.BlockSpec((_BR, _D), lambda i: (i, 0)),
            pl.BlockSpec((_BR, _D), lambda i: (i, 0)),
            pl.BlockSpec((_BR, _D), lambda i, _nb=nb: (i + _nb, 0)),
            pl.BlockSpec((_BR, 1), lambda i: (i, 0)),
            pl.BlockSpec((_BR, 1), lambda i: (i, 0)),
        ],
        out_specs=pl.BlockSpec((_BR, _D), lambda i: (i, 0)),
        out_shape=jax.ShapeDtypeStruct((_N, _D), _F32),
    )(x1, g, g, w0, w1)


def kernel(x, ln1_g, ln1_b, c_attn_w, c_attn_b, c_proj_w, c_proj_b,
           ln2_g, ln2_b, w_g, c_fc, fc_bias, c_proj_e, proj_bias):
    nc, ns = _sc_meshinfo()
    nw = nc * ns

    x2d = x.reshape(_N, _D)
    q, k, v = _ln_qkv(x2d, ln1_g, ln1_b, c_attn_w, c_attn_b)
    y = _attention(q, k, v)

    x1, h2, logits = _proj_ln2_logits(y, x2d, c_proj_w, c_proj_b,
                                      ln2_g, ln2_b, w_g)

    rt, cnt = _router(logits, nw)
    slots = jnp.concatenate([rt[:, 0], rt[:, 1]]).astype(jnp.int32)
    slots = slots.reshape(nw, _NPAIR // nw)
    w0 = rt[:, 2:3]
    w1 = rt[:, 3:4]

    disp = _dispatch_sc(h2, slots)
    o_flat = _expert_mlps(disp, c_fc, fc_bias, c_proj_e, proj_bias,
                          _SLOTS + nw)
    g = _gather_sc(o_flat, slots)
    out = _combine(x1, g, w0, w1)
    return out.reshape(_B, _T, _D)


# speedup vs baseline: 1.3497x; 1.0091x over previous
import functools
import math

import jax
import jax.numpy as jnp
from jax import lax
from jax.experimental import pallas as pl
from jax.experimental.pallas import tpu as pltpu
from jax.experimental.pallas import tpu_sc as plsc

_B, _T, _D, _H, _E, _K = 1, 2048, 768, 12, 8, 2
_DFF = 4 * _D
_N = _B * _T
_cc = math.floor(_K * 1.25 * _N / _E)
_cc += _cc % 2
_CAP = max(_cc, 4)
_HD = _D // _H
_SLOTS = _E * _CAP
_NPAIR = _K * _N

_BQ = 1024
_BR = 512
_F32 = jnp.float32


def _f32dot(a, b):
    return jnp.dot(a, b, preferred_element_type=_F32)


def _f32dot_nt(a, b):
    return lax.dot_general(a, b, (((1,), (1,)), ((), ())),
                           preferred_element_type=_F32)


def _gelu(x):
    return 0.5 * x * (1.0 + lax.erf(x * (1.0 / math.sqrt(2.0))))


def _ln_qkv_body(x_ref, g_ref, b_ref, w_ref, bias_ref, q_ref, k_ref, v_ref):
    x = x_ref[...]
    m = jnp.mean(x, axis=-1, keepdims=True)
    c = x - m
    v = jnp.mean(c * c, axis=-1, keepdims=True)
    h = c / jnp.sqrt(v + 1e-5) * g_ref[...] + b_ref[...]
    qkv = _f32dot_nt(h, w_ref[...]) + bias_ref[...]
    for hh in range(_H):
        q_ref[hh] = qkv[:, hh * _HD:(hh + 1) * _HD]
        k_ref[hh] = qkv[:, _D + hh * _HD:_D + (hh + 1) * _HD]
        v_ref[hh] = qkv[:, 2 * _D + hh * _HD:2 * _D + (hh + 1) * _HD]


def _ln_qkv(x2d, g, b, w, bias):
    hs = jax.ShapeDtypeStruct((_H, _T, _HD), _F32)
    return pl.pallas_call(
        _ln_qkv_body,
        grid=(_N // _BR,),
        in_specs=[
            pl.BlockSpec((_BR, _D), lambda i: (i, 0)),
            pl.BlockSpec((1, _D), lambda i: (0, 0)),
            pl.BlockSpec((1, _D), lambda i: (0, 0)),
            pl.BlockSpec((3 * _D, _D), lambda i: (0, 0)),
            pl.BlockSpec((1, 3 * _D), lambda i: (0, 0)),
        ],
        out_specs=[pl.BlockSpec((_H, _BR, _HD), lambda i: (0, i, 0))] * 3,
        out_shape=[hs, hs, hs],
    )(x2d, g.reshape(1, _D), b.reshape(1, _D), w, bias.reshape(1, 3 * _D))


def _attn_body(q_ref, k_ref, v_ref, o_ref):
    qi = pl.program_id(1)
    q = q_ref[0]
    scale = 1.0 / math.sqrt(_HD)

    def tile(ki, carry, masked):
        m, l, acc = carry
        k = k_ref[0, pl.ds(ki * _BQ, _BQ), :]
        s = _f32dot_nt(q, k) * scale
        if masked:
            rows = lax.broadcasted_iota(jnp.int32, (_BQ, _BQ), 0)
            cols = lax.broadcasted_iota(jnp.int32, (_BQ, _BQ), 1)
            s = jnp.where(rows >= cols, s, -jnp.inf)
        mn = jnp.maximum(m, jnp.max(s, axis=1, keepdims=True))
        p = jnp.exp(s - mn)
        alpha = jnp.exp(m - mn)
        l2 = l * alpha + jnp.sum(p, axis=1, keepdims=True)
        v = v_ref[0, pl.ds(ki * _BQ, _BQ), :]
        acc2 = acc * alpha + _f32dot(p, v)
        return mn, l2, acc2

    m0 = jnp.full((_BQ, 1), -jnp.inf, _F32)
    l0 = jnp.zeros((_BQ, 1), _F32)
    a0 = jnp.zeros((_BQ, _HD), _F32)
    carry = lax.fori_loop(0, qi, lambda ki, c: tile(ki, c, False),
                          (m0, l0, a0))
    m, l, acc = tile(qi, carry, True)
    o_ref[0] = acc / l


def _attention(q, k, v):
    return pl.pallas_call(
        _attn_body,
        grid=(_H, _T // _BQ),
        in_specs=[
            pl.BlockSpec((1, _BQ, _HD), lambda h, i: (h, i, 0)),
            pl.BlockSpec((1, _T, _HD), lambda h, i: (h, 0, 0)),
            pl.BlockSpec((1, _T, _HD), lambda h, i: (h, 0, 0)),
        ],
        out_specs=pl.BlockSpec((1, _BQ, _HD), lambda h, i: (h, i, 0)),
        out_shape=jax.ShapeDtypeStruct((_H, _T, _HD), _F32),
    )(q, k, v)


def _proj_body(nworkers, y_ref, x_ref, w_ref, b_ref, g2_ref, b2_ref, wg_ref,
               x1_ref, h2_ref, rt_ref, cnt_ref,
               lg_ref, oh0_ref, oh1_ref, c0_ref, c1_ref):
    i = pl.program_id(0)
    y = jnp.concatenate([y_ref[hh] for hh in range(_H)], axis=1)
    x1 = x_ref[...] + _f32dot_nt(y, w_ref[...]) + b_ref[...]
    x1_ref[...] = x1
    m = jnp.mean(x1, axis=-1, keepdims=True)
    c = x1 - m
    v = jnp.mean(c * c, axis=-1, keepdims=True)
    h2 = c / jnp.sqrt(v + 1e-5) * g2_ref[...] + b2_ref[...]
    h2_ref[...] = h2
    lg_ref[pl.ds(i * _BR, _BR), :] = _f32dot_nt(h2, wg_ref[...])

    @pl.when(i == _N // _BR - 1)
    def _():
        _router_logic(nworkers, lg_ref, rt_ref, cnt_ref,
                      oh0_ref, oh1_ref, c0_ref, c1_ref)


def _proj_ln2_router(y, x2d, w, b, g2, b2, wg, nworkers):
    return pl.pallas_call(
        functools.partial(_proj_body, nworkers),
        grid=(_N // _BR,),
        in_specs=[
            pl.BlockSpec((_H, _BR, _HD), lambda i: (0, i, 0)),
            pl.BlockSpec((_BR, _D), lambda i: (i, 0)),
            pl.BlockSpec((_D, _D), lambda i: (0, 0)),
            pl.BlockSpec((1, _D), lambda i: (0, 0)),
            pl.BlockSpec((1, _D), lambda i: (0, 0)),
            pl.BlockSpec((1, _D), lambda i: (0, 0)),
            pl.BlockSpec((_E, _D), lambda i: (0, 0)),
        ],
        out_specs=[
            pl.BlockSpec((_BR, _D), lambda i: (i, 0)),
            pl.BlockSpec((_BR, _D), lambda i: (i, 0)),
            pl.BlockSpec((_N, 8), lambda i: (0, 0)),
            pl.BlockSpec((1, _E), lambda i: (0, 0)),
        ],
        out_shape=[
            jax.ShapeDtypeStruct((_N, _D), _F32),
            jax.ShapeDtypeStruct((_N, _D), _F32),
            jax.ShapeDtypeStruct((_N, 8), _F32),
            jax.ShapeDtypeStruct((1, _E), jnp.int32),
        ],
        scratch_shapes=[pltpu.VMEM((_N, _E), _F32) for _ in range(5)],
    )(y, x2d, w, b.reshape(1, _D), g2.reshape(1, _D), b2.reshape(1, _D), wg)


_RCH = 256


def _router_logic(nworkers, lg_ref, rt_ref, cnt_ref, oh0_ref, oh1_ref, c0_ref, c1_ref):
    l = lg_ref[...]
    eidx = lax.broadcasted_iota(jnp.int32, (_N, _E), 1)
    m0 = jnp.max(l, axis=1, keepdims=True)
    e0 = jnp.min(jnp.where(l == m0, eidx, _E), axis=1, keepdims=True)
    oh0 = (eidx == e0)
    lm = jnp.where(oh0, -jnp.inf, l)
    m1 = jnp.max(lm, axis=1, keepdims=True)
    e1 = jnp.min(jnp.where(lm == m1, eidx, _E), axis=1, keepdims=True)
    oh1 = (eidx == e1)
    t = jnp.exp(m1 - m0)
    w0 = 1.0 / (1.0 + t)
    w1 = t * w0
    oh0_ref[...] = oh0.astype(_F32)
    oh1_ref[...] = oh1.astype(_F32)

    tril = (lax.broadcasted_iota(jnp.int32, (_RCH, _RCH), 0)
            >= lax.broadcasted_iota(jnp.int32, (_RCH, _RCH), 1)).astype(_F32)

    def scan(oh_ref, cum_ref):
        def chunk(i, tot):
            oh = oh_ref[pl.ds(i * _RCH, _RCH), :]
            cum_ref[pl.ds(i * _RCH, _RCH), :] = _f32dot(tril, oh) + tot
            return tot + jnp.sum(oh, axis=0, keepdims=True)
        return lax.fori_loop(0, _N // _RCH, chunk, jnp.zeros((1, _E), _F32))

    tot0 = scan(oh0_ref, c0_ref)
    tot1 = scan(oh1_ref, c1_ref)
    cnt_ref[...] = jnp.minimum(tot0 + tot1, float(_CAP)).astype(jnp.int32)

    oh0f = oh0_ref[...]
    oh1f = oh1_ref[...]
    p0 = jnp.sum(oh0f * c0_ref[...], axis=1, keepdims=True) - 1.0
    p1 = jnp.sum(oh1f * (c1_ref[...] + tot0), axis=1, keepdims=True) - 1.0

    ppw = _NPAIR // nworkers
    nidx = lax.broadcasted_iota(jnp.int32, (_N, 1), 0)
    dump0 = (_SLOTS + nidx // ppw).astype(_F32)
    dump1 = (_SLOTS + (_N + nidx) // ppw).astype(_F32)
    keep0 = p0 < _CAP
    keep1 = p1 < _CAP
    e0f = e0.astype(_F32)
    e1f = e1.astype(_F32)
    d0 = jnp.where(keep0, e0f * _CAP + p0, dump0)
    d1 = jnp.where(keep1, e1f * _CAP + p1, dump1)
    w0 = jnp.where(keep0, w0, 0.0)
    w1 = jnp.where(keep1, w1, 0.0)
    z = jnp.zeros((_N, 4), _F32)
    rt_ref[...] = jnp.concatenate([d0, d1, w0, w1, z], axis=1)


def _sc_meshinfo():
    info = plsc.get_sparse_core_info()
    return info.num_cores, info.num_subcores


def _dispatch_sc(h2, slots):
    nc, ns = _sc_meshinfo()
    nw = nc * ns
    ppw = _NPAIR // nw
    mesh = plsc.VectorSubcoreMesh(core_axis_name="c", subcore_axis_name="s")

    @functools.partial(
        pl.kernel, mesh=mesh,
        out_type=jax.ShapeDtypeStruct((_SLOTS + nw, _D), _F32),
        scratch_types=[
            pltpu.VMEM((ppw,), jnp.int32),
            pltpu.VMEM((ppw, _D), _F32),
        ],
    )
    def k(h2_hbm, slots_hbm, out_hbm, idx_v, rows_v):
        wid = lax.axis_index("s") * nc + lax.axis_index("c")
        tok = (wid * ppw) % _N
        pltpu.sync_copy(slots_hbm.at[wid], idx_v)
        pltpu.sync_copy(h2_hbm.at[pl.ds(tok, ppw)], rows_v)
        pltpu.sync_copy(rows_v, out_hbm.at[idx_v])

    return k(h2, slots)


def _gather_sc(o_flat, slots):
    nc, ns = _sc_meshinfo()
    nw = nc * ns
    ppw = _NPAIR // nw
    mesh = plsc.VectorSubcoreMesh(core_axis_name="c", subcore_axis_name="s")

    @functools.partial(
        pl.kernel, mesh=mesh,
        out_type=jax.ShapeDtypeStruct((_NPAIR, _D), _F32),
        scratch_types=[
            pltpu.VMEM((ppw,), jnp.int32),
            pltpu.VMEM((ppw, _D), _F32),
            pltpu.SemaphoreType.DMA,
        ],
    )
    def k(o_hbm, slots_hbm, out_hbm, idx_v, rows_v, sem):
        wid = lax.axis_index("s") * nc + lax.axis_index("c")
        pltpu.sync_copy(slots_hbm.at[wid], idx_v)
        pltpu.async_copy(o_hbm.at[idx_v], rows_v, sem).wait()
        pltpu.sync_copy(rows_v, out_hbm.at[pl.ds(wid * ppw, ppw)])

    return k(o_flat, slots)


def _mlp_body(disp_ref, wfc_ref, bfc_ref, wpr_ref, bpr_ref, o_ref):
    a = _gelu(_f32dot(disp_ref[...], wfc_ref[0]) + bfc_ref[0])
    o_ref[...] = _f32dot(a, wpr_ref[0]) + bpr_ref[0]


def _expert_mlps(disp, c_fc, fc_bias, c_proj_e, proj_bias, nrows):
    return pl.pallas_call(
        _mlp_body,
        grid=(_E,),
        in_specs=[
            pl.BlockSpec((_CAP, _D), lambda e: (e, 0)),
            pl.BlockSpec((1, _D, _DFF), lambda e: (e, 0, 0)),
            pl.BlockSpec((1, 1, _DFF), lambda e: (e, 0, 0)),
            pl.BlockSpec((1, _DFF, _D), lambda e: (e, 0, 0)),
            pl.BlockSpec((1, 1, _D), lambda e: (e, 0, 0)),
        ],
        out_specs=pl.BlockSpec((_CAP, _D), lambda e: (e, 0)),
        out_shape=jax.ShapeDtypeStruct((nrows, _D), _F32),
    )(disp, c_fc, fc_bias, c_proj_e, proj_bias)


def _combine_body(x1_ref, g0_ref, g1_ref, w0_ref, w1_ref, o_ref):
    w0 = w0_ref[...]
    w1 = w1_ref[...]
    o_ref[...] = (x1_ref[...]
                  + jnp.where(w0 == 0.0, 0.0, w0 * g0_ref[...])
                  + jnp.where(w1 == 0.0, 0.0, w1 * g1_ref[...]))


def _combine(x1, g, w0, w1):
    nb = _N // _BR
    return pl.pallas_call(
        _combine_body,
        grid=(nb,),
        in_specs=[
            pl.BlockSpec((_BR, _D), lambda i: (i, 0)),
            pl.BlockSpec((_BR, _D), lambda i: (i, 0)),
            pl.BlockSpec((_BR, _D), lambda i, _nb=nb: (i + _nb, 0)),
            pl.BlockSpec((_BR, 1), lambda i: (i, 0)),
            pl.BlockSpec((_BR, 1), lambda i: (i, 0)),
        ],
        out_specs=pl.BlockSpec((_BR, _D), lambda i: (i, 0)),
        out_shape=jax.ShapeDtypeStruct((_N, _D), _F32),
    )(x1, g, g, w0, w1)


def kernel(x, ln1_g, ln1_b, c_attn_w, c_attn_b, c_proj_w, c_proj_b,
           ln2_g, ln2_b, w_g, c_fc, fc_bias, c_proj_e, proj_bias):
    nc, ns = _sc_meshinfo()
    nw = nc * ns

    x2d = x.reshape(_N, _D)
    q, k, v = _ln_qkv(x2d, ln1_g, ln1_b, c_attn_w, c_attn_b)
    y = _attention(q, k, v)

    x1, h2, rt, cnt = _proj_ln2_router(y, x2d, c_proj_w, c_proj_b,
                                       ln2_g, ln2_b, w_g, nw)
    del cnt
    slots = jnp.concatenate([rt[:, 0], rt[:, 1]]).astype(jnp.int32)
    slots = slots.reshape(nw, _NPAIR // nw)
    w0 = rt[:, 2:3]
    w1 = rt[:, 3:4]

    disp = _dispatch_sc(h2, slots)
    o_flat = _expert_mlps(disp, c_fc, fc_bias, c_proj_e, proj_bias,
                          _SLOTS + nw)
    g = _gather_sc(o_flat, slots)
    out = _combine(x1, g, w0, w1)
    return out.reshape(_B, _T, _D)
